# initial kernel scaffold (unmeasured)
import jax
import jax.numpy as jnp
from jax import lax
from jax.experimental import pallas as pl
from jax.experimental.pallas import tpu as pltpu


def kernel(
    t,
):
    def body(*refs):
        pass

    out_shape = jax.ShapeDtypeStruct(..., jnp.float32)
    return pl.pallas_call(body, out_shape=out_shape)(...)


# baseline (device time: 64231 ns/iter reference)
import jax
import jax.numpy as jnp
from jax import lax
from jax.experimental import pallas as pl
from jax.experimental.pallas import tpu as pltpu

N_DEV = 32
LOG2_N = 5


def kernel(t):
    m, n = t.shape

    def body(t_ref, out_ref, comm_ref, send_sems, recv_sems):
        my = lax.axis_index("i")

        out_ref[...] = t_ref[...]

        for k in range(LOG2_N):
            partner = my ^ (1 << k)
            rdma = pltpu.make_async_remote_copy(
                src_ref=out_ref,
                dst_ref=comm_ref.at[k],
                send_sem=send_sems.at[k],
                recv_sem=recv_sems.at[k],
                device_id=(partner,),
                device_id_type=pl.DeviceIdType.MESH,
            )
            rdma.start()
            rdma.wait()
            out_ref[...] = out_ref[...] + comm_ref[k]

        s = out_ref[...]
        r = jnp.maximum(s, 0.0)
        out_ref[...] = jnp.tanh(s) * s * s + r * r * r

    return pl.pallas_call(
        body,
        out_shape=jax.ShapeDtypeStruct((m, n), jnp.float32),
        in_specs=[pl.BlockSpec(memory_space=pltpu.VMEM)],
        out_specs=pl.BlockSpec(memory_space=pltpu.VMEM),
        scratch_shapes=[
            pltpu.VMEM((LOG2_N, m, n), jnp.float32),
            pltpu.SemaphoreType.DMA((LOG2_N,)),
            pltpu.SemaphoreType.DMA((LOG2_N,)),
        ],
    )(t)


# device time: 26543 ns/iter; 2.4199x vs baseline; 2.4199x over previous
import jax
import jax.numpy as jnp
from jax import lax
from jax.experimental import pallas as pl
from jax.experimental.pallas import tpu as pltpu

N_DEV = 32


def kernel(t):
    m, n = t.shape
    rows = m // N_DEV

    def body(
        t_ref,
        out_ref,
        stage1,
        comm1,
        stage2,
        comm2,
        send1,
        recv1,
        send2,
        recv2,
    ):
        my = lax.axis_index("i")

        stage1[...] = t_ref[...].astype(jnp.bfloat16)
        rdma1 = []
        for o in range(1, N_DEV):
            d = (my + o) % N_DEV
            r = pltpu.make_async_remote_copy(
                src_ref=stage1.at[pl.ds(d * rows, rows), :],
                dst_ref=comm1.at[o],
                send_sem=send1.at[o],
                recv_sem=recv1.at[o],
                device_id=(d,),
                device_id_type=pl.DeviceIdType.MESH,
            )
            r.start()
            rdma1.append(r)

        acc = t_ref[pl.ds(my * rows, rows), :]
        for o in range(1, N_DEV):
            rdma1[o - 1].wait()
            acc = acc + comm1[o].astype(jnp.float32)

        rpos = jnp.maximum(acc, 0.0)
        y = jnp.tanh(acc) * acc * acc + rpos * rpos * rpos
        out_ref[pl.ds(my * rows, rows), :] = y
        stage2[...] = y.astype(jnp.bfloat16)

        rdma2 = []
        for o in range(1, N_DEV):
            d = (my + o) % N_DEV
            r = pltpu.make_async_remote_copy(
                src_ref=stage2,
                dst_ref=comm2.at[o],
                send_sem=send2.at[o],
                recv_sem=recv2.at[o],
                device_id=(d,),
                device_id_type=pl.DeviceIdType.MESH,
            )
            r.start()
            rdma2.append(r)

        for o in range(1, N_DEV):
            rdma2[o - 1].wait()
            s = (my - o) % N_DEV
            out_ref[pl.ds(s * rows, rows), :] = comm2[o].astype(jnp.float32)

    return pl.pallas_call(
        body,
        out_shape=jax.ShapeDtypeStruct((m, n), jnp.float32),
        in_specs=[pl.BlockSpec(memory_space=pltpu.VMEM)],
        out_specs=pl.BlockSpec(memory_space=pltpu.VMEM),
        scratch_shapes=[
            pltpu.VMEM((m, n), jnp.bfloat16),
            pltpu.VMEM((N_DEV, rows, n), jnp.bfloat16),
            pltpu.VMEM((rows, n), jnp.bfloat16),
            pltpu.VMEM((N_DEV, rows, n), jnp.bfloat16),
            pltpu.SemaphoreType.DMA((N_DEV,)),
            pltpu.SemaphoreType.DMA((N_DEV,)),
            pltpu.SemaphoreType.DMA((N_DEV,)),
            pltpu.SemaphoreType.DMA((N_DEV,)),
        ],
    )(t)


# device time: 22779 ns/iter; 2.8197x vs baseline; 1.1652x over previous
import jax
import jax.numpy as jnp
from jax import lax
from jax.experimental import pallas as pl
from jax.experimental.pallas import tpu as pltpu

N_DEV = 32


def kernel(t):
    m, n = t.shape
    rows = m // N_DEV

    def body(
        t_ref,
        out_ref,
        stage1,
        comm1,
        stage2,
        comm2,
        send1,
        recv1,
        send2,
        recv2,
    ):
        my = lax.axis_index("i")

        barrier_sem = pltpu.get_barrier_semaphore()
        for o in range(1, N_DEV):
            pl.semaphore_signal(
                barrier_sem,
                inc=1,
                device_id=((my + o) % N_DEV,),
                device_id_type=pl.DeviceIdType.MESH,
            )

        stage1[...] = t_ref[...].astype(jnp.bfloat16)
        pl.semaphore_wait(barrier_sem, N_DEV - 1)
        rdma1 = []
        for o in range(1, N_DEV):
            d = (my + o) % N_DEV
            r = pltpu.make_async_remote_copy(
                src_ref=stage1.at[pl.ds(d * rows, rows), :],
                dst_ref=comm1.at[o],
                send_sem=send1.at[o],
                recv_sem=recv1.at[o],
                device_id=(d,),
                device_id_type=pl.DeviceIdType.MESH,
            )
            r.start()
            rdma1.append(r)

        acc = t_ref[pl.ds(my * rows, rows), :]
        for o in range(1, N_DEV):
            rdma1[o - 1].wait()
            acc = acc + comm1[o].astype(jnp.float32)

        rpos = jnp.maximum(acc, 0.0)
        y = jnp.tanh(acc) * acc * acc + rpos * rpos * rpos
        out_ref[pl.ds(my * rows, rows), :] = y
        stage2[...] = y.astype(jnp.bfloat16)

        rdma2 = []
        for o in range(1, N_DEV):
            d = (my + o) % N_DEV
            r = pltpu.make_async_remote_copy(
                src_ref=stage2,
                dst_ref=comm2.at[o],
                send_sem=send2.at[o],
                recv_sem=recv2.at[o],
                device_id=(d,),
                device_id_type=pl.DeviceIdType.MESH,
            )
            r.start()
            rdma2.append(r)

        for o in range(1, N_DEV):
            rdma2[o - 1].wait()
            s = (my - o) % N_DEV
            out_ref[pl.ds(s * rows, rows), :] = comm2[o].astype(jnp.float32)

    return pl.pallas_call(
        body,
        out_shape=jax.ShapeDtypeStruct((m, n), jnp.float32),
        in_specs=[pl.BlockSpec(memory_space=pltpu.VMEM)],
        out_specs=pl.BlockSpec(memory_space=pltpu.VMEM),
        scratch_shapes=[
            pltpu.VMEM((m, n), jnp.bfloat16),
            pltpu.VMEM((N_DEV, rows, n), jnp.bfloat16),
            pltpu.VMEM((rows, n), jnp.bfloat16),
            pltpu.VMEM((N_DEV, rows, n), jnp.bfloat16),
            pltpu.SemaphoreType.DMA((N_DEV,)),
            pltpu.SemaphoreType.DMA((N_DEV,)),
            pltpu.SemaphoreType.DMA((N_DEV,)),
            pltpu.SemaphoreType.DMA((N_DEV,)),
        ],
        compiler_params=pltpu.CompilerParams(collective_id=0),
    )(t)
